# Initial kernel scaffold; baseline (speedup 1.0000x reference)
import jax
import jax.numpy as jnp
from jax.experimental import pallas as pl


def kernel(node_attr, edge_attr, edge_index, edge_contact_attr, edge_contact_index, W1, b1, W2, b2):
    raise NotImplementedError("write your pallas kernel here")


# speedup vs baseline: 1.0587x; 1.0587x over previous
import functools

import jax
import jax.numpy as jnp
from jax.experimental import pallas as pl
from jax.experimental.pallas import tpu as pltpu
from jax.experimental.pallas import tpu_sc as plsc

N_NODES = 10000
N_EDGES = 320000
D_FEAT = 128
D_EDGE = 16
D_HID = 128
D_OUT = 16

_W = 128
_NPAD = 327680
_GRID = _NPAD // _W
_B3 = 2000
_LANES = 16


def _node_proj(node_attr, w1ab):
    def body(n_ref, w_ref, p_ref, q_ref):
        n = n_ref[...]
        p_ref[...] = jnp.dot(n, w_ref[:D_FEAT, :],
                             preferred_element_type=jnp.float32)
        q_ref[...] = jnp.dot(n, w_ref[D_FEAT:, :],
                             preferred_element_type=jnp.float32)

    return pl.pallas_call(
        body,
        out_shape=[
            jax.ShapeDtypeStruct((N_NODES, D_HID), jnp.float32),
            jax.ShapeDtypeStruct((N_NODES, D_HID), jnp.float32),
        ],
    )(node_attr, w1ab)


def _sc_gather_add(p, q, sidx, ridx):
    mesh = plsc.VectorSubcoreMesh(core_axis_name="c", subcore_axis_name="s")

    @functools.partial(
        pl.kernel,
        out_type=jax.ShapeDtypeStruct((_NPAD, D_HID), jnp.float32),
        mesh=mesh,
        scratch_types=[pltpu.VMEM((_W, D_HID), jnp.float32)],
    )
    def k(p_hbm, q_hbm, si_hbm, ri_hbm, o_hbm, gp_vmem):
        def body(si_vmem, ri_vmem, o_vmem):
            pltpu.sync_copy(p_hbm.at[si_vmem.at[0]], gp_vmem)
            pltpu.sync_copy(q_hbm.at[ri_vmem.at[0]], o_vmem)

            @pl.loop(0, _W)
            def _(r):
                for c in range(0, D_HID, _LANES):
                    sl = pl.ds(c, _LANES)
                    o_vmem.at[r, sl][...] += gp_vmem.at[r, sl][...]

        pltpu.emit_pipeline(
            body,
            grid=(_GRID,),
            in_specs=[
                pl.BlockSpec((1, _W), lambda i: (0, i)),
                pl.BlockSpec((1, _W), lambda i: (0, i)),
            ],
            out_specs=[pl.BlockSpec((_W, D_HID), lambda i: (i, 0))],
            core_axis_name=("c", "s"),
            dimension_semantics=(pltpu.PARALLEL,),
        )(si_hbm, ri_hbm, o_hbm)

    return k(p, q, sidx, ridx)


def _epilogue(g, edge_attr, w1c, b1, w2, b2):
    def body(g_ref, e_ref, w1c_ref, b1_ref, w2_ref, b2_ref, o_ref):
        pre = (g_ref[...]
               + jnp.dot(e_ref[...], w1c_ref[...],
                         preferred_element_type=jnp.float32)
               + b1_ref[...])
        h = jnp.maximum(pre, 0.0)
        o_ref[...] = (jnp.dot(h, w2_ref[...],
                              preferred_element_type=jnp.float32)
                      + b2_ref[...])

    return pl.pallas_call(
        body,
        grid=(N_EDGES // _B3,),
        in_specs=[
            pl.BlockSpec((_B3, D_HID), lambda i: (i, 0)),
            pl.BlockSpec((_B3, D_EDGE), lambda i: (i, 0)),
            pl.BlockSpec((D_EDGE, D_HID), lambda i: (0, 0)),
            pl.BlockSpec((1, D_HID), lambda i: (0, 0)),
            pl.BlockSpec((D_HID, D_OUT), lambda i: (0, 0)),
            pl.BlockSpec((1, D_OUT), lambda i: (0, 0)),
        ],
        out_specs=pl.BlockSpec((_B3, D_OUT), lambda i: (i, 0)),
        out_shape=jax.ShapeDtypeStruct((N_EDGES, D_OUT), jnp.float32),
    )(g, edge_attr, w1c, b1, w2, b2)


def kernel(node_attr, edge_attr, edge_index, edge_contact_attr,
           edge_contact_index, W1, b1, W2, b2):
    idx = edge_index.astype(jnp.int32)
    idx = jnp.pad(idx, ((0, 0), (0, _NPAD - N_EDGES)))
    sidx = idx[0].reshape(1, _NPAD)
    ridx = idx[1].reshape(1, _NPAD)

    p, q = _node_proj(node_attr, W1[: 2 * D_FEAT, :])
    g = _sc_gather_add(p, q, sidx, ridx)
    out = _epilogue(g, edge_attr, W1[2 * D_FEAT:, :],
                    b1.reshape(1, D_HID), W2, b2.reshape(1, D_OUT))
    return (node_attr, out, edge_index, edge_contact_attr, edge_contact_index)


# speedup vs baseline: 1.8312x; 1.7297x over previous
import functools

import jax
import jax.numpy as jnp
from jax.experimental import pallas as pl
from jax.experimental.pallas import tpu as pltpu
from jax.experimental.pallas import tpu_sc as plsc

N_NODES = 10000
N_EDGES = 320000
D_FEAT = 128
D_EDGE = 16
D_HID = 128
D_OUT = 16

_W = 128
_NPAD = 327680
_GRID = _NPAD // _W
_B3 = 2000
_LANES = 16


def _node_proj(node_attr, w1ab):
    def body(n_ref, w_ref, p_ref, q_ref):
        n = n_ref[...]
        p_ref[...] = jnp.dot(n, w_ref[:D_FEAT, :],
                             preferred_element_type=jnp.float32)
        q_ref[...] = jnp.dot(n, w_ref[D_FEAT:, :],
                             preferred_element_type=jnp.float32)

    return pl.pallas_call(
        body,
        out_shape=[
            jax.ShapeDtypeStruct((N_NODES, D_HID), jnp.float32),
            jax.ShapeDtypeStruct((N_NODES, D_HID), jnp.float32),
        ],
    )(node_attr, w1ab)


_NWORK = 32
_WIN_PER = _GRID // _NWORK


def _sc_gather_add(p, q, sidx, ridx):
    mesh = plsc.VectorSubcoreMesh(core_axis_name="c", subcore_axis_name="s")

    @functools.partial(
        pl.kernel,
        out_type=jax.ShapeDtypeStruct((_NPAD, D_HID), jnp.float32),
        mesh=mesh,
        scratch_types=[
            pltpu.VMEM((2, _W), jnp.int32),
            pltpu.VMEM((2, _W), jnp.int32),
            pltpu.VMEM((2, _W, D_HID), jnp.float32),
            pltpu.VMEM((2, _W, D_HID), jnp.float32),
            pltpu.SemaphoreType.DMA,
            pltpu.SemaphoreType.DMA,
            pltpu.SemaphoreType.DMA,
            pltpu.SemaphoreType.DMA,
            pltpu.SemaphoreType.DMA,
            pltpu.SemaphoreType.DMA,
        ],
    )
    def k(p_hbm, q_hbm, si_hbm, ri_hbm, o_hbm, si_v, ri_v, gp_v, gq_v,
          s_i0, s_i1, s_g0, s_g1, s_o0, s_o1):
        sem_i = (s_i0, s_i1)
        sem_g = (s_g0, s_g1)
        sem_o = (s_o0, s_o1)
        wid = jax.lax.axis_index("s") * 2 + jax.lax.axis_index("c")
        w0 = wid * _WIN_PER

        def fire_idx(j, b):
            pltpu.make_async_copy(si_hbm.at[w0 + j], si_v.at[b], sem_i[b]).start()
            pltpu.make_async_copy(ri_hbm.at[w0 + j], ri_v.at[b], sem_i[b]).start()

        def wait_idx(j, b):
            pltpu.make_async_copy(si_hbm.at[w0 + j], si_v.at[b], sem_i[b]).wait()
            pltpu.make_async_copy(ri_hbm.at[w0 + j], ri_v.at[b], sem_i[b]).wait()

        def fire_gathers(b):
            pltpu.make_async_copy(p_hbm.at[si_v.at[b]], gp_v.at[b], sem_g[b]).start()
            pltpu.make_async_copy(q_hbm.at[ri_v.at[b]], gq_v.at[b], sem_g[b]).start()

        def wait_gathers(b):
            pltpu.make_async_copy(p_hbm.at[si_v.at[b]], gp_v.at[b], sem_g[b]).wait()
            pltpu.make_async_copy(q_hbm.at[ri_v.at[b]], gq_v.at[b], sem_g[b]).wait()

        def out_copy(j, b):
            return pltpu.make_async_copy(
                gq_v.at[b], o_hbm.at[pl.ds((w0 + j) * _W, _W), :], sem_o[b])

        fire_idx(0, 0)
        wait_idx(0, 0)
        fire_gathers(0)
        fire_idx(1, 1)

        @pl.loop(0, _WIN_PER, step=2)
        def _(j0):
            for b in (0, 1):
                j = j0 + b
                nb = 1 - b

                @pl.when(jnp.logical_and(j >= 1, j + 1 < _WIN_PER))
                def _():
                    out_copy(j - 1, nb).wait()

                @pl.when(j + 1 < _WIN_PER)
                def _():
                    wait_idx(j + 1, nb)
                    fire_gathers(nb)

                wait_gathers(b)

                @pl.when(j + 2 < _WIN_PER)
                def _():
                    fire_idx(j + 2, b)

                @pl.loop(0, _W)
                def _(r):
                    for c in range(0, D_HID, _LANES):
                        sl = pl.ds(c, _LANES)
                        gq_v.at[b, r, sl][...] += gp_v.at[b, r, sl][...]

                out_copy(j, b).start()

        out_copy(_WIN_PER - 2, 0).wait()
        out_copy(_WIN_PER - 1, 1).wait()

    return k(p, q, sidx, ridx)


def _epilogue(g, edge_attr, w1c, b1, w2, b2):
    def body(g_ref, e_ref, w1c_ref, b1_ref, w2_ref, b2_ref, o_ref):
        pre = (g_ref[...]
               + jnp.dot(e_ref[...], w1c_ref[...],
                         preferred_element_type=jnp.float32)
               + b1_ref[...])
        h = jnp.maximum(pre, 0.0)
        o_ref[...] = (jnp.dot(h, w2_ref[...],
                              preferred_element_type=jnp.float32)
                      + b2_ref[...])

    return pl.pallas_call(
        body,
        grid=(N_EDGES // _B3,),
        in_specs=[
            pl.BlockSpec((_B3, D_HID), lambda i: (i, 0)),
            pl.BlockSpec((_B3, D_EDGE), lambda i: (i, 0)),
            pl.BlockSpec((D_EDGE, D_HID), lambda i: (0, 0)),
            pl.BlockSpec((1, D_HID), lambda i: (0, 0)),
            pl.BlockSpec((D_HID, D_OUT), lambda i: (0, 0)),
            pl.BlockSpec((1, D_OUT), lambda i: (0, 0)),
        ],
        out_specs=pl.BlockSpec((_B3, D_OUT), lambda i: (i, 0)),
        out_shape=jax.ShapeDtypeStruct((N_EDGES, D_OUT), jnp.float32),
    )(g, edge_attr, w1c, b1, w2, b2)


def kernel(node_attr, edge_attr, edge_index, edge_contact_attr,
           edge_contact_index, W1, b1, W2, b2):
    idx = edge_index.astype(jnp.int32)
    idx = jnp.pad(idx, ((0, 0), (0, _NPAD - N_EDGES)))
    sidx = idx[0].reshape(_GRID, _W)
    ridx = idx[1].reshape(_GRID, _W)

    p, q = _node_proj(node_attr, W1[: 2 * D_FEAT, :])
    g = _sc_gather_add(p, q, sidx, ridx)
    out = _epilogue(g, edge_attr, W1[2 * D_FEAT:, :],
                    b1.reshape(1, D_HID), W2, b2.reshape(1, D_OUT))
    return (node_attr, out, edge_index, edge_contact_attr, edge_contact_index)
